# baseline (speedup 1.0000x reference)
import jax
import jax.numpy as jnp
import numpy as np
from jax.experimental import pallas as pl
from jax.experimental.pallas import tpu as pltpu

_EPS = 0.001
_TOPK = 128
_INT_MIN = np.int32(-2147483648)
_INT_MAX = np.int32(2147483647)


def _ordered_key(x):
    u = jax.lax.bitcast_convert_type(x, jnp.int32)
    return jnp.where(u >= 0, u, jnp.bitwise_xor(jnp.bitwise_not(u), _INT_MIN))


def _key_to_f32(k):
    u = jnp.where(k >= 0, k, jnp.bitwise_not(jnp.bitwise_xor(k, _INT_MIN)))
    return jax.lax.bitcast_convert_type(u, jnp.float32)


def _select_cutoff_key(key):
    lo, hi = jnp.int32(_INT_MIN), jnp.int32(_INT_MAX)
    for _ in range(32):
        mid = (lo >> 1) + (hi >> 1) + (lo & hi & 1)
        cnt = jnp.sum((key > mid).astype(jnp.int32))
        take_hi = cnt <= _TOPK
        lo = jnp.where(take_hi, lo, mid + 1)
        hi = jnp.where(take_hi, mid, hi)
    return lo


def _make_body(B, L, D, NL):
    dc = D // NL
    lc = L // NL

    def _body(q_ref, k_ref, v_ref, o_ref, acc_ref, sc_ref, scc_ref, mrow_ref):
        s = pl.program_id(0)
        n = pl.program_id(1)
        par = s % 2
        par1 = (s + 1) % 2

        for p in (0, 1):
            @pl.when(jnp.logical_and(s < B, par == p))
            def _(p=p):
                qp = jax.nn.relu(q_ref[0]) + _EPS
                ones = jnp.ones((1, L), jnp.float32)
                col = jax.lax.dot_general(
                    ones, qp, (((1,), (0,)), ((), ())),
                    preferred_element_type=jnp.float32)
                acc_ref[p, :, pl.ds(n * dc, dc)] = col

        for p in (0, 1):
            @pl.when(jnp.logical_and(
                jnp.logical_and(s >= 1, s <= B), par1 == p))
            def _(p=p):
                kp = jax.nn.relu(k_ref[0]) + _EPS
                sv = jax.lax.dot_general(
                    acc_ref[p], kp, (((1,), (1,)), ((), ())),
                    preferred_element_type=jnp.float32)
                sc_ref[p, :, pl.ds(n * lc, lc)] = sv
                rows = lc // 128
                scc_ref[p, pl.ds(n * rows, rows), :] = sv.reshape(rows, 128)

        for p in (0, 1):
            @pl.when(jnp.logical_and(
                jnp.logical_and(s >= 2, n == 0), par == p))
            def _(p=p):
                cut = _select_cutoff_key(_ordered_key(scc_ref[p]))
                m = (sc_ref[p] > _key_to_f32(cut)).astype(jnp.float32)
                mrow_ref[...] = jnp.reshape(m, (L, 1))

        @pl.when(s >= 2)
        def _():
            m = mrow_ref[pl.ds(n * lc, lc), :]
            o_ref[0] = v_ref[0] * m

    return _body


def kernel(queries, keys, values):
    B, L, D = queries.shape
    NL = 2
    dc = D // NL
    lc = L // NL

    def q_idx(s, n):
        return (jnp.minimum(s, B - 1), 0, jnp.where(s < B, n, NL - 1))

    def k_idx(s, n):
        b = jnp.clip(s - 1, 0, B - 1)
        c = jnp.where(s < 1, 0, jnp.where(s <= B, n, NL - 1))
        return (b, c, 0)

    def v_idx(s, n):
        return (jnp.clip(s - 2, 0, B - 1), jnp.where(s >= 2, n, 0), 0)

    return pl.pallas_call(
        _make_body(B, L, D, NL),
        grid=(B + 2, NL),
        in_specs=[
            pl.BlockSpec((1, L, dc), q_idx),
            pl.BlockSpec((1, lc, D), k_idx),
            pl.BlockSpec((1, lc, D), v_idx),
        ],
        out_specs=pl.BlockSpec((1, lc, D), v_idx),
        out_shape=jax.ShapeDtypeStruct((B, L, D), jnp.float32),
        scratch_shapes=[
            pltpu.VMEM((2, 1, D), jnp.float32),
            pltpu.VMEM((2, 1, L), jnp.float32),
            pltpu.VMEM((2, L // 128, 128), jnp.float32),
            pltpu.VMEM((L, 1), jnp.float32),
        ],
    )(queries, keys, values)
